# baseline (speedup 1.0000x reference)
import jax
import jax.numpy as jnp
from jax.experimental import pallas as pl
from jax.experimental.pallas import tpu as pltpu

VOCAB = 32000
ROW_BLK = 16


def _onehot_kernel(ids_ref, out_ref):
    col = jax.lax.broadcasted_iota(jnp.int32, (ROW_BLK, VOCAB), 1)
    nid = jax.lax.rem(ids_ref[...] + 1, VOCAB)
    out_ref[...] = jnp.where(col == nid, 20.0, -20.0)


def kernel(input_ids):
    B, S = input_ids.shape
    rows = B * S
    ids = input_ids.reshape(rows, 1).astype(jnp.int32)
    out = pl.pallas_call(
        _onehot_kernel,
        grid=(rows // ROW_BLK,),
        in_specs=[pl.BlockSpec((ROW_BLK, 1), lambda i: (i, 0))],
        out_specs=pl.BlockSpec((ROW_BLK, VOCAB), lambda i: (i, 0)),
        out_shape=jax.ShapeDtypeStruct((rows, VOCAB), jnp.float32),
        compiler_params=pltpu.CompilerParams(
            dimension_semantics=("arbitrary",),
        ),
    )(ids)
    return out.reshape(B, S, VOCAB)


# speedup vs baseline: 1.2755x; 1.2755x over previous
import jax
import jax.numpy as jnp
from jax.experimental import pallas as pl
from jax.experimental.pallas import tpu as pltpu

VOCAB = 32000
ROW_BLK = 32
NTILES = 32
NBUF = 8


def _onehot_kernel(ids_ref, out_hbm, vbuf, sem):
    col = jax.lax.broadcasted_iota(jnp.int32, (ROW_BLK, VOCAB), 1)
    for i in range(NTILES):
        slot = i % NBUF
        if i >= NBUF:
            pltpu.make_async_copy(
                vbuf.at[slot],
                out_hbm.at[pl.ds((i - NBUF) * ROW_BLK, ROW_BLK), :],
                sem.at[slot],
            ).wait()
        nid = jax.lax.rem(ids_ref[pl.ds(i * ROW_BLK, ROW_BLK), :] + 1, VOCAB)
        vbuf[slot] = jnp.where(col == nid, 20.0, -20.0)
        pltpu.make_async_copy(
            vbuf.at[slot],
            out_hbm.at[pl.ds(i * ROW_BLK, ROW_BLK), :],
            sem.at[slot],
        ).start()
    for i in range(NTILES - NBUF, NTILES):
        pltpu.make_async_copy(
            vbuf.at[i % NBUF],
            out_hbm.at[pl.ds(i * ROW_BLK, ROW_BLK), :],
            sem.at[i % NBUF],
        ).wait()


def kernel(input_ids):
    B, S = input_ids.shape
    rows = B * S
    ids = input_ids.reshape(rows, 1).astype(jnp.int32)
    out = pl.pallas_call(
        _onehot_kernel,
        in_specs=[pl.BlockSpec(memory_space=pltpu.MemorySpace.VMEM)],
        out_specs=pl.BlockSpec(memory_space=pltpu.MemorySpace.HBM),
        out_shape=jax.ShapeDtypeStruct((rows, VOCAB), jnp.float32),
        scratch_shapes=[
            pltpu.VMEM((NBUF, ROW_BLK, VOCAB), jnp.float32),
            pltpu.SemaphoreType.DMA((NBUF,)),
        ],
    )(ids)
    return out.reshape(B, S, VOCAB)


# speedup vs baseline: 1.3716x; 1.0754x over previous
import jax
import jax.numpy as jnp
from jax.experimental import pallas as pl
from jax.experimental.pallas import tpu as pltpu

VOCAB = 32000
ROW_BLK = 32


def _onehot_kernel(ids_ref, out_ref):
    col = jax.lax.broadcasted_iota(jnp.int32, (ROW_BLK, VOCAB), 1)
    nid = jax.lax.rem(ids_ref[...] + 1, VOCAB)
    out_ref[...] = jnp.where(col == nid, 20.0, -20.0)


def kernel(input_ids):
    B, S = input_ids.shape
    rows = B * S
    ids = input_ids.reshape(rows, 1).astype(jnp.int32)
    out = pl.pallas_call(
        _onehot_kernel,
        grid=(rows // ROW_BLK,),
        in_specs=[pl.BlockSpec((ROW_BLK, 1), lambda i: (i, 0))],
        out_specs=pl.BlockSpec((ROW_BLK, VOCAB), lambda i: (i, 0)),
        out_shape=jax.ShapeDtypeStruct((rows, VOCAB), jnp.float32),
        compiler_params=pltpu.CompilerParams(
            dimension_semantics=("parallel",),
        ),
    )(ids)
    return out.reshape(B, S, VOCAB)
